# baseline (speedup 1.0000x reference)
import functools

import jax
import jax.numpy as jnp
from jax import lax
from jax.experimental import pallas as pl
from jax.experimental.pallas import tpu as pltpu
from jax.experimental.pallas import tpu_sc as plsc

B, C, N, K = 4, 64, 1024, 9
NODES = B * N
NW = 32
NPW = NODES // NW
RB = 1024
CH = 32

STEP_EDGE = [[0, 1], [2, 3, 4], [5, 6, 7, 8], [9, 10, 11, 12, 13]]
STATE_FUTURE = [[0, 2, 5, 9], [1, 3, 6, 10], [4, 7, 11], [8, 12], [13]]


def _dot(a, b):
    return lax.dot(a, b, precision=lax.Precision.HIGHEST,
                   preferred_element_type=jnp.float32)


def _ntab(n_f):
    return (n_f + 3) // 4


RT = 64


def _knn_body(xr_ref, xa_ref, idx_ref):
    b = pl.program_id(0)
    xr = xr_ref[0]
    xa = xa_ref[0]
    g = lax.dot_general(xr, xa, (((1,), (1,)), ((), ())),
                        preferred_element_type=jnp.float32)
    xxr = jnp.sum(xr * xr, axis=1)
    xxa = jnp.sum(xa * xa, axis=1)
    dist0 = xxr[:, None] - 2.0 * g + xxa[None, :]
    lane = lax.broadcasted_iota(jnp.int32, (RT, N), 1)
    lane16 = lax.broadcasted_iota(jnp.int32, (RT, 16), 1)
    acc0 = jnp.zeros((RT, 16), jnp.int32)

    def body(k, carry):
        dist, acc = carry
        idx = jnp.argmin(dist, axis=1).astype(jnp.int32)
        acc = jnp.where(lane16 == k, (idx + b * N)[:, None], acc)
        dist = jnp.where(lane == idx[:, None], jnp.float32(jnp.inf), dist)
        return dist, acc

    _, acc = lax.fori_loop(0, K, body, (dist0, acc0))
    idx_ref[0] = acc


def _knn(x_bnc):
    call = pl.pallas_call(
        _knn_body,
        grid=(B, N // RT),
        in_specs=[pl.BlockSpec((1, RT, C), lambda b, r: (b, r, 0)),
                  pl.BlockSpec((1, N, C), lambda b, r: (b, 0, 0))],
        out_specs=pl.BlockSpec((1, RT, 16), lambda b, r: (b, r, 0)),
        out_shape=jax.ShapeDtypeStruct((B, N, 16), jnp.int32),
    )
    return call(x_bnc, x_bnc)


def _prep_body(x0_ref, x1_ref, Wp0_ref, gp0_ref, bp0_ref,
               Wp1_ref, gp1_ref, bp1_ref,
               Wav0_ref, bav0_ref, Wav1_ref, bav1_ref, *outs):
    n0 = len(STATE_FUTURE[0])
    nt = _ntab(n0)
    per = n0 + nt
    for s_i, (x_ref, Wp_ref, gp_ref, bp_ref, Wav_ref, bav_ref) in enumerate((
            (x0_ref, Wp0_ref, gp0_ref, bp0_ref, Wav0_ref, bav0_ref),
            (x1_ref, Wp1_ref, gp1_ref, bp1_ref, Wav1_ref, bav1_ref))):
        h = jnp.maximum(_dot(x_ref[...], Wp_ref[...]) * gp_ref[0]
                        + bp_ref[0], 0.0)
        av = _dot(h, Wav_ref[...]) + bav_ref[0]
        base = s_i * per
        for t in range(n0):
            outs[base + t][...] = av[:, 64 * t:64 * t + 64]
        for p in range(nt):
            outs[base + n0 + p][...] = av[:, 64 * n0 + 256 * p:
                                          64 * n0 + 256 * p + 256
                                          ].astype(jnp.bfloat16)


def _prep(x0, x1, Wp0, gp0, bp0, Wp1, gp1, bp1, Wav0, bav0, Wav1, bav1):
    n0 = len(STATE_FUTURE[0])
    nt = _ntab(n0)
    wdt = 64 * n0 + 256 * nt
    row = pl.BlockSpec((RB, C), lambda r: (r, 0))
    rowt = pl.BlockSpec((RB, 256), lambda r: (r, 0))
    full = lambda s: pl.BlockSpec(s, lambda r: (0,) * len(s))
    call = pl.pallas_call(
        _prep_body,
        grid=(NODES // RB,),
        in_specs=[row, row,
                  full((C, C)), full((1, C)), full((1, C)),
                  full((C, C)), full((1, C)), full((1, C)),
                  full((C, wdt)), full((1, wdt)),
                  full((C, wdt)), full((1, wdt))],
        out_specs=([row] * n0 + [rowt] * nt) * 2,
        out_shape=([jax.ShapeDtypeStruct((NODES, C), jnp.float32)] * n0
                   + [jax.ShapeDtypeStruct((NODES, 256), jnp.bfloat16)] * nt
                   ) * 2,
    )
    return call(x0, x1, Wp0, gp0, bp0, Wp1, gp1, bp1, Wav0, bav0, Wav1, bav1)


def _make_combine(n_e, n_f, mmap, n_mt):
    nt = _ntab(n_f)

    def body(*refs):
        As = refs[:n_e]
        Mt = refs[n_e:n_e + n_mt]
        p = n_e + n_mt
        if n_f:
            Wav_ref, bav_ref = refs[p], refs[p + 1]
            p += 2
        h_ref = refs[p]
        outs = refs[p + 1:]
        s = None
        for t, a_ref in enumerate(As):
            tab, q = mmap[t]
            m = Mt[tab][:, 64 * q:64 * q + 64].astype(jnp.float32)
            v = jnp.maximum(a_ref[...] + m, 0.0)
            s = v if s is None else s + v
        h_ref[...] = s
        if n_f:
            av = _dot(s, Wav_ref[...]) + bav_ref[0]
            for t in range(n_f):
                outs[t][...] = av[:, 64 * t:64 * t + 64]
            for p2 in range(nt):
                outs[n_f + p2][...] = av[:, 64 * n_f + 256 * p2:
                                         64 * n_f + 256 * p2 + 256
                                         ].astype(jnp.bfloat16)

    row = pl.BlockSpec((RB, C), lambda r: (r, 0))
    rowt = pl.BlockSpec((RB, 256), lambda r: (r, 0))
    wdt = 64 * n_f + 256 * nt
    in_specs = [row] * n_e + [rowt] * n_mt
    if n_f:
        in_specs += [pl.BlockSpec((C, wdt), lambda r: (0, 0)),
                     pl.BlockSpec((1, wdt), lambda r: (0, 0))]
    call = pl.pallas_call(
        body,
        grid=(NODES // RB,),
        in_specs=in_specs,
        out_specs=[row] * (1 + n_f) + [rowt] * nt,
        out_shape=([jax.ShapeDtypeStruct((NODES, C), jnp.float32)] * (1 + n_f)
                   + [jax.ShapeDtypeStruct((NODES, 256), jnp.bfloat16)] * nt),
    )
    return call


def _make_gather_max(n_t):
    mesh = plsc.VectorSubcoreMesh(core_axis_name="c", subcore_axis_name="s",
                                  num_cores=2, num_subcores=16)

    n_ch = NPW // CH

    @functools.partial(
        pl.kernel,
        out_type=[jax.ShapeDtypeStruct((NODES, 128), jnp.int32)] * n_t,
        mesh=mesh,
        scratch_types=[
            pltpu.VMEM((K, NPW), jnp.int32),
            pltpu.VMEM((2, K * CH, 128), jnp.int32),
            pltpu.VMEM((2, CH, 128), jnp.int32),
            pltpu.SemaphoreType.DMA,
            pltpu.SemaphoreType.DMA,
            pltpu.SemaphoreType.DMA,
        ],
    )
    def k(idx_hbm, *refs):
        v_hbms = refs[:n_t]
        out_hbms = refs[n_t:2 * n_t]
        idx_v, buf, out_v, semA, semB, semW = refs[2 * n_t:]
        sems = (semA, semB)
        wid = lax.axis_index("s") * 2 + lax.axis_index("c")
        base = wid * NPW
        pltpu.sync_copy(idx_hbm.at[wid], idx_v)

        chunks = [(t, q) for t in range(n_t) for q in range(n_ch)]

        def fire(s):
            t, q = chunks[s]
            par = s % 2
            return [pltpu.async_copy(
                v_hbms[t].at[idx_v.at[kk, pl.ds(q * CH, CH)]],
                buf.at[par, pl.ds(kk * CH, CH)], sems[par])
                for kk in range(K)]

        pend = fire(0)
        wr = [None, None]
        for s, (t, q) in enumerate(chunks):
            par = s % 2
            nxt = fire(s + 1) if s + 1 < len(chunks) else []
            for cp in pend:
                cp.wait()
            pend = nxt
            if wr[par] is not None:
                wr[par].wait()

            def body(i, _, par=par):
                himask = jnp.int32(-65536)
                maxi = jnp.int32(0x7FFFFFFF)

                def t(x):
                    return x ^ ((x >> 31) & maxi)

                def keys(x):
                    return t(x << 16), t(x & himask)

                for g2 in range(8):
                    sl = pl.ds(g2 * 16, 16)
                    alo, ahi = keys(buf[par, i, sl])
                    for kk in range(1, K):
                        blo, bhi = keys(buf[par, kk * CH + i, sl])
                        alo = jnp.maximum(alo, blo)
                        ahi = jnp.maximum(ahi, bhi)
                    out_v[par, i, sl] = (
                        (t(ahi) & himask)
                        | ((t(alo) >> 16) & jnp.int32(0xFFFF)))
                return 0

            lax.fori_loop(0, CH, body, 0)
            wr[par] = pltpu.async_copy(
                out_v.at[par], out_hbms[t].at[pl.ds(base + q * CH, CH)],
                semW)
        for w in wr:
            if w is not None:
                w.wait()

    return k


def kernel(s0, s1, weights, selected_idxs, x_0, curstage_selected_idxs,
           curstage_candidate_flags, Wp0, gp0, bp0, Wp1, gp1, bp1,
           Wops, gops, bops):
    f32 = jnp.float32
    x0 = jnp.transpose(s0[:, :, :, 0], (0, 2, 1)).reshape(NODES, C)
    x1 = jnp.transpose(s1[:, :, :, 0], (0, 2, 1)).reshape(NODES, C)

    gate = (selected_idxs != 0).astype(f32)
    Wi, Wd = Wops[:, :C, :], Wops[:, C:, :]
    gg = gops * gate[:, None]
    Wa = (Wi - Wd) * gg[:, None, :]
    Wv = Wd * gg[:, None, :]
    ba = bops * gate[:, None]

    def pack(es):
        cols = [Wa[e] for e in es]
        bias = [ba[e] for e in es]
        for p in range(_ntab(len(es))):
            quad = es[4 * p:4 * p + 4]
            blk = [Wv[e] for e in quad]
            blk += [jnp.zeros((C, C), f32)] * (4 - len(quad))
            cols.append(jnp.concatenate(blk, axis=1))
            bias.append(jnp.zeros((256,), f32))
        return jnp.concatenate(cols, axis=1), jnp.concatenate(bias)[None, :]

    nnk = _knn(x0.reshape(B, N, C))
    idx3 = jnp.transpose(nnk[:, :, :K].reshape(NW, NPW, K), (0, 2, 1))

    Wav0, bav0 = pack(STATE_FUTURE[0])
    Wav1, bav1 = pack(STATE_FUTURE[1])
    prep_out = _prep(x0, x1, Wp0, gp0[None, :], bp0[None, :],
                     Wp1, gp1[None, :], bp1[None, :], Wav0, bav0, Wav1, bav1)
    A, V, M = {}, {}, {}
    n0 = len(STATE_FUTURE[0])
    nt0 = _ntab(n0)
    for s_i in (0, 1):
        es = STATE_FUTURE[s_i]
        base = s_i * (n0 + nt0)
        for t, e in enumerate(es):
            A[e] = prep_out[base + t]
            V[e] = (prep_out[base + n0 + t // 4], t % 4)

    def run_gather(es):
        tabs32 = [lax.bitcast_convert_type(
            V[e][0].reshape(NODES, 128, 2), jnp.int32) for e in es[::4]]
        ms = _make_gather_max(len(tabs32))(idx3, *tabs32)
        mbf = None
        for t, e in enumerate(es):
            if t % 4 == 0:
                mbf = lax.bitcast_convert_type(
                    ms[t // 4], jnp.bfloat16).reshape(NODES, 256)
            M[e] = (mbf, t % 4)

    run_gather(STATE_FUTURE[0] + STATE_FUTURE[1])

    h_out = []
    for i in range(4):
        edges = STEP_EDGE[i]
        fut = STATE_FUTURE[2 + i] if i < 3 else []
        mt_refs, mt_pos, mmap = [], {}, []
        for e in edges:
            arr, half = M[e]
            key = id(arr)
            if key not in mt_pos:
                mt_pos[key] = len(mt_refs)
                mt_refs.append(arr)
            mmap.append((mt_pos[key], half))
        args = [A[e] for e in edges] + mt_refs
        if fut:
            wavf, bavf = pack(fut)
            args += [wavf, bavf]
        res = _make_combine(len(edges), len(fut), tuple(mmap),
                            len(mt_refs))(*args)
        h_out.append(res[0])
        for t, e in enumerate(fut):
            A[e] = res[1 + t]
            V[e] = (res[1 + len(fut) + t // 4], t % 4)
        if fut:
            run_gather(fut)

    outs = [jnp.transpose(h.reshape(B, N, C), (0, 2, 1))[..., None]
            for h in h_out]
    return jnp.concatenate(outs, axis=1)


# speedup vs baseline: 1.5381x; 1.5381x over previous
import functools

import jax
import jax.numpy as jnp
from jax import lax
from jax.experimental import pallas as pl
from jax.experimental.pallas import tpu as pltpu
from jax.experimental.pallas import tpu_sc as plsc

B, C, N, K = 4, 64, 1024, 9
NODES = B * N
NW = 32
NPW = NODES // NW
RB = 1024
CH = 32

STEP_EDGE = [[0, 1], [2, 3, 4], [5, 6, 7, 8], [9, 10, 11, 12, 13]]
STATE_FUTURE = [[0, 2, 5, 9], [1, 3, 6, 10], [4, 7, 11], [8, 12], [13]]


def _dot(a, b):
    return lax.dot(a, b, precision=lax.Precision.HIGHEST,
                   preferred_element_type=jnp.float32)


def _ntab(n_f):
    return (n_f + 1) // 2


RT = 64


def _knn_body(xr_ref, xa_ref, idx_ref):
    b = pl.program_id(0)
    xr = xr_ref[0]
    xa = xa_ref[0]
    g = lax.dot_general(xr, xa, (((1,), (1,)), ((), ())),
                        preferred_element_type=jnp.float32)
    xxr = jnp.sum(xr * xr, axis=1)
    xxa = jnp.sum(xa * xa, axis=1)
    dist0 = xxr[:, None] - 2.0 * g + xxa[None, :]
    lane = lax.broadcasted_iota(jnp.int32, (RT, N), 1)
    lane16 = lax.broadcasted_iota(jnp.int32, (RT, 16), 1)
    acc0 = jnp.zeros((RT, 16), jnp.int32)

    def body(k, carry):
        dist, acc = carry
        idx = jnp.argmin(dist, axis=1).astype(jnp.int32)
        acc = jnp.where(lane16 == k, (idx + b * N)[:, None], acc)
        dist = jnp.where(lane == idx[:, None], jnp.float32(jnp.inf), dist)
        return dist, acc

    _, acc = lax.fori_loop(0, K, body, (dist0, acc0))
    idx_ref[0] = acc


def _knn(x_bnc):
    call = pl.pallas_call(
        _knn_body,
        grid=(B, N // RT),
        in_specs=[pl.BlockSpec((1, RT, C), lambda b, r: (b, r, 0)),
                  pl.BlockSpec((1, N, C), lambda b, r: (b, 0, 0))],
        out_specs=pl.BlockSpec((1, RT, 16), lambda b, r: (b, r, 0)),
        out_shape=jax.ShapeDtypeStruct((B, N, 16), jnp.int32),
    )
    return call(x_bnc, x_bnc)


def _prep_body(x0_ref, x1_ref, Wp0_ref, gp0_ref, bp0_ref,
               Wp1_ref, gp1_ref, bp1_ref,
               Wav0_ref, bav0_ref, Wav1_ref, bav1_ref, *outs):
    n0 = len(STATE_FUTURE[0])
    nt = _ntab(n0)
    per = n0 + nt
    for s_i, (x_ref, Wp_ref, gp_ref, bp_ref, Wav_ref, bav_ref) in enumerate((
            (x0_ref, Wp0_ref, gp0_ref, bp0_ref, Wav0_ref, bav0_ref),
            (x1_ref, Wp1_ref, gp1_ref, bp1_ref, Wav1_ref, bav1_ref))):
        h = jnp.maximum(_dot(x_ref[...], Wp_ref[...]) * gp_ref[0]
                        + bp_ref[0], 0.0)
        av = _dot(h, Wav_ref[...]) + bav_ref[0]
        base = s_i * per
        for t in range(n0):
            outs[base + t][...] = av[:, 64 * t:64 * t + 64]
        for p in range(nt):
            outs[base + n0 + p][...] = av[:, 64 * n0 + 128 * p:
                                          64 * n0 + 128 * p + 128]


def _prep(x0, x1, Wp0, gp0, bp0, Wp1, gp1, bp1, Wav0, bav0, Wav1, bav1):
    n0 = len(STATE_FUTURE[0])
    nt = _ntab(n0)
    wdt = 64 * n0 + 128 * nt
    row = pl.BlockSpec((RB, C), lambda r: (r, 0))
    rowt = pl.BlockSpec((RB, 128), lambda r: (r, 0))
    full = lambda s: pl.BlockSpec(s, lambda r: (0,) * len(s))
    call = pl.pallas_call(
        _prep_body,
        grid=(NODES // RB,),
        in_specs=[row, row,
                  full((C, C)), full((1, C)), full((1, C)),
                  full((C, C)), full((1, C)), full((1, C)),
                  full((C, wdt)), full((1, wdt)),
                  full((C, wdt)), full((1, wdt))],
        out_specs=([row] * n0 + [rowt] * nt) * 2,
        out_shape=([jax.ShapeDtypeStruct((NODES, C), jnp.float32)] * n0
                   + [jax.ShapeDtypeStruct((NODES, 128), jnp.float32)] * nt
                   ) * 2,
    )
    return call(x0, x1, Wp0, gp0, bp0, Wp1, gp1, bp1, Wav0, bav0, Wav1, bav1)


def _make_combine(n_e, n_f, mmap, n_mt):
    nt = _ntab(n_f)

    def body(*refs):
        As = refs[:n_e]
        Mt = refs[n_e:n_e + n_mt]
        p = n_e + n_mt
        if n_f:
            Wav_ref, bav_ref = refs[p], refs[p + 1]
            p += 2
        h_ref = refs[p]
        outs = refs[p + 1:]
        s = None
        for t, a_ref in enumerate(As):
            tab, half = mmap[t]
            m = Mt[tab][:, 64 * half:64 * half + 64]
            v = jnp.maximum(a_ref[...] + m, 0.0)
            s = v if s is None else s + v
        h_ref[...] = s
        if n_f:
            av = _dot(s, Wav_ref[...]) + bav_ref[0]
            for t in range(n_f):
                outs[t][...] = av[:, 64 * t:64 * t + 64]
            for p2 in range(nt):
                outs[n_f + p2][...] = av[:, 64 * n_f + 128 * p2:
                                         64 * n_f + 128 * p2 + 128]

    row = pl.BlockSpec((RB, C), lambda r: (r, 0))
    rowt = pl.BlockSpec((RB, 128), lambda r: (r, 0))
    wdt = 64 * n_f + 128 * nt
    in_specs = [row] * n_e + [rowt] * n_mt
    if n_f:
        in_specs += [pl.BlockSpec((C, wdt), lambda r: (0, 0)),
                     pl.BlockSpec((1, wdt), lambda r: (0, 0))]
    call = pl.pallas_call(
        body,
        grid=(NODES // RB,),
        in_specs=in_specs,
        out_specs=[row] * (1 + n_f) + [rowt] * nt,
        out_shape=([jax.ShapeDtypeStruct((NODES, C), jnp.float32)] * (1 + n_f)
                   + [jax.ShapeDtypeStruct((NODES, 128), jnp.float32)] * nt),
    )
    return call


def _make_gather_max(n_t):
    mesh = plsc.VectorSubcoreMesh(core_axis_name="c", subcore_axis_name="s",
                                  num_cores=2, num_subcores=16)

    n_ch = NPW // CH

    @functools.partial(
        pl.kernel,
        out_type=[jax.ShapeDtypeStruct((NODES, 128), jnp.float32)] * n_t,
        mesh=mesh,
        scratch_types=[
            pltpu.VMEM((K, NPW), jnp.int32),
            pltpu.VMEM((2, K * CH, 128), jnp.float32),
            pltpu.VMEM((2, CH, 128), jnp.float32),
            pltpu.SemaphoreType.DMA,
            pltpu.SemaphoreType.DMA,
            pltpu.SemaphoreType.DMA,
        ],
    )
    def k(idx_hbm, *refs):
        v_hbms = refs[:n_t]
        out_hbms = refs[n_t:2 * n_t]
        idx_v, buf, out_v, semA, semB, semW = refs[2 * n_t:]
        sems = (semA, semB)
        wid = lax.axis_index("s") * 2 + lax.axis_index("c")
        base = wid * NPW
        pltpu.sync_copy(idx_hbm.at[wid], idx_v)

        chunks = [(t, q) for t in range(n_t) for q in range(n_ch)]

        def fire(s):
            t, q = chunks[s]
            par = s % 2
            return [pltpu.async_copy(
                v_hbms[t].at[idx_v.at[kk, pl.ds(q * CH, CH)]],
                buf.at[par, pl.ds(kk * CH, CH)], sems[par])
                for kk in range(K)]

        pend = fire(0)
        wr = [None, None]
        for s, (t, q) in enumerate(chunks):
            par = s % 2
            nxt = fire(s + 1) if s + 1 < len(chunks) else []
            for cp in pend:
                cp.wait()
            pend = nxt
            if wr[par] is not None:
                wr[par].wait()

            def body(i, _, par=par):
                for g2 in range(8):
                    sl = pl.ds(g2 * 16, 16)
                    acc = buf[par, i, sl]
                    for kk in range(1, K):
                        acc = jnp.maximum(acc, buf[par, kk * CH + i, sl])
                    out_v[par, i, sl] = acc
                return 0

            lax.fori_loop(0, CH, body, 0)
            wr[par] = pltpu.async_copy(
                out_v.at[par], out_hbms[t].at[pl.ds(base + q * CH, CH)],
                semW)
        for w in wr:
            if w is not None:
                w.wait()

    return k


def kernel(s0, s1, weights, selected_idxs, x_0, curstage_selected_idxs,
           curstage_candidate_flags, Wp0, gp0, bp0, Wp1, gp1, bp1,
           Wops, gops, bops):
    f32 = jnp.float32
    x0 = jnp.transpose(s0[:, :, :, 0], (0, 2, 1)).reshape(NODES, C)
    x1 = jnp.transpose(s1[:, :, :, 0], (0, 2, 1)).reshape(NODES, C)

    gate = (selected_idxs != 0).astype(f32)
    Wi, Wd = Wops[:, :C, :], Wops[:, C:, :]
    gg = gops * gate[:, None]
    Wa = (Wi - Wd) * gg[:, None, :]
    Wv = Wd * gg[:, None, :]
    ba = bops * gate[:, None]

    def pack(es):
        cols = [Wa[e] for e in es]
        bias = [ba[e] for e in es]
        for p in range(_ntab(len(es))):
            pair = es[2 * p:2 * p + 2]
            blk = [Wv[e] for e in pair]
            if len(pair) == 1:
                blk.append(jnp.zeros((C, C), f32))
            cols.append(jnp.concatenate(blk, axis=1))
            bias.append(jnp.zeros((128,), f32))
        return jnp.concatenate(cols, axis=1), jnp.concatenate(bias)[None, :]

    nnk = _knn(x0.reshape(B, N, C))
    idx3 = jnp.transpose(nnk[:, :, :K].reshape(NW, NPW, K), (0, 2, 1))

    Wav0, bav0 = pack(STATE_FUTURE[0])
    Wav1, bav1 = pack(STATE_FUTURE[1])
    prep_out = _prep(x0, x1, Wp0, gp0[None, :], bp0[None, :],
                     Wp1, gp1[None, :], bp1[None, :], Wav0, bav0, Wav1, bav1)
    A, V, M = {}, {}, {}
    n0 = len(STATE_FUTURE[0])
    nt0 = _ntab(n0)
    for s_i in (0, 1):
        es = STATE_FUTURE[s_i]
        base = s_i * (n0 + nt0)
        for t, e in enumerate(es):
            A[e] = prep_out[base + t]
            V[e] = (prep_out[base + n0 + t // 2], t % 2)

    def run_gather(es):
        tabs = []
        for e in es[::2]:
            tabs.append(V[e][0])
        ms = _make_gather_max(len(tabs))(idx3, *tabs)
        for t, e in enumerate(es):
            M[e] = (ms[t // 2], t % 2)

    run_gather(STATE_FUTURE[0] + STATE_FUTURE[1])

    h_out = []
    for i in range(4):
        edges = STEP_EDGE[i]
        fut = STATE_FUTURE[2 + i] if i < 3 else []
        mt_refs, mt_pos, mmap = [], {}, []
        for e in edges:
            arr, half = M[e]
            key = id(arr)
            if key not in mt_pos:
                mt_pos[key] = len(mt_refs)
                mt_refs.append(arr)
            mmap.append((mt_pos[key], half))
        args = [A[e] for e in edges] + mt_refs
        if fut:
            wavf, bavf = pack(fut)
            args += [wavf, bavf]
        res = _make_combine(len(edges), len(fut), tuple(mmap),
                            len(mt_refs))(*args)
        h_out.append(res[0])
        for t, e in enumerate(fut):
            A[e] = res[1 + t]
            V[e] = (res[1 + len(fut) + t // 2], t % 2)
        if fut:
            run_gather(fut)

    outs = [jnp.transpose(h.reshape(B, N, C), (0, 2, 1))[..., None]
            for h in h_out]
    return jnp.concatenate(outs, axis=1)
